# Initial kernel scaffold; baseline (speedup 1.0000x reference)
import jax
import jax.numpy as jnp
from jax.experimental import pallas as pl


def kernel(input_ids, table):
    raise NotImplementedError("write your pallas kernel here")


# speedup vs baseline: 4.8066x; 4.8066x over previous
import functools

import jax
import jax.numpy as jnp
from jax import lax
from jax.experimental import pallas as pl
from jax.experimental.pallas import tpu as pltpu
from jax.experimental.pallas import tpu_sc as plsc


def _make_gather(N, V, D, NC, NS):
    NW = NC * NS
    CHUNK = 128
    K = 8
    GROUP = K * CHUNK
    assert N % (NW * GROUP) == 0
    n_per_w = N // NW
    n_groups = n_per_w // GROUP

    mesh = plsc.VectorSubcoreMesh(core_axis_name="c", subcore_axis_name="s")

    @functools.partial(
        pl.kernel,
        mesh=mesh,
        out_type=jax.ShapeDtypeStruct((N, D), jnp.float32),
        scratch_types=[
            pltpu.VMEM((K, CHUNK), jnp.int32),
            pltpu.VMEM((GROUP, D), jnp.float32),
            pltpu.SemaphoreType.DMA,
        ],
        compiler_params=pltpu.CompilerParams(use_tc_tiling_on_sc=False),
    )
    def gather_kernel(idx_hbm, table_hbm, out_hbm, idx_v, rows_v, sem):
        wid = lax.axis_index("s") * NC + lax.axis_index("c")
        row_base = wid * (n_per_w // CHUNK)

        def group_body(g, carry):
            grow = row_base + g * K
            pltpu.sync_copy(idx_hbm.at[pl.ds(grow, K)], idx_v)
            copies = []
            for j in range(K):
                copies.append(
                    pltpu.async_copy(
                        table_hbm.at[idx_v.at[j]],
                        rows_v.at[pl.ds(j * CHUNK, CHUNK)],
                        sem,
                    )
                )
            for cp in copies:
                cp.wait()
            pltpu.sync_copy(
                rows_v, out_hbm.at[pl.ds(grow * CHUNK, GROUP)]
            )
            return carry

        lax.fori_loop(0, n_groups, group_body, 0)

    return gather_kernel


def kernel(input_ids, table):
    B, H = input_ids.shape
    V, D = table.shape
    N = B * H
    info = plsc.get_sparse_core_info()
    NC, NS = info.num_cores, info.num_subcores
    idx2d = input_ids.reshape(N // 128, 128)
    gather = _make_gather(N, V, D, NC, NS)
    out = gather(idx2d, table)
    return out.reshape(B, H, D)


# speedup vs baseline: 4.9509x; 1.0300x over previous
import functools

import jax
import jax.numpy as jnp
from jax import lax
from jax.experimental import pallas as pl
from jax.experimental.pallas import tpu as pltpu
from jax.experimental.pallas import tpu_sc as plsc


def _make_gather(N, V, D, NC, NS):
    NW = NC * NS
    CHUNK = 128
    K = 8
    GROUP = K * CHUNK
    n_per_w = N // NW
    n_groups = n_per_w // GROUP
    assert N % (NW * GROUP) == 0 and n_groups % 2 == 0
    n_pairs = n_groups // 2

    mesh = plsc.VectorSubcoreMesh(core_axis_name="c", subcore_axis_name="s")

    @functools.partial(
        pl.kernel,
        mesh=mesh,
        out_type=jax.ShapeDtypeStruct((N, D), jnp.float32),
        scratch_types=[
            pltpu.VMEM((K, CHUNK), jnp.int32),
            pltpu.VMEM((K, CHUNK), jnp.int32),
            pltpu.VMEM((GROUP, D), jnp.float32),
            pltpu.VMEM((GROUP, D), jnp.float32),
            pltpu.SemaphoreType.DMA,
            pltpu.SemaphoreType.DMA,
            pltpu.SemaphoreType.DMA,
            pltpu.SemaphoreType.DMA,
        ],
        compiler_params=pltpu.CompilerParams(use_tc_tiling_on_sc=False),
    )
    def gather_kernel(
        idx_hbm, table_hbm, out_hbm,
        idx_v0, idx_v1, rows_v0, rows_v1, gsem0, gsem1, osem0, osem1,
    ):
        wid = lax.axis_index("s") * NC + lax.axis_index("c")
        row_base = wid * (n_per_w // CHUNK)
        idx_v = (idx_v0, idx_v1)
        rows_v = (rows_v0, rows_v1)
        gsem = (gsem0, gsem1)
        osem = (osem0, osem1)

        def fire_group(g, b):
            grow = row_base + g * K
            pltpu.sync_copy(idx_hbm.at[pl.ds(grow, K)], idx_v[b])
            for j in range(K):
                pltpu.async_copy(
                    table_hbm.at[idx_v[b].at[j]],
                    rows_v[b].at[pl.ds(j * CHUNK, CHUNK)],
                    gsem[b],
                )

        def drain_gathers(b):
            for j in range(K):
                pltpu.make_async_copy(
                    table_hbm.at[idx_v[b].at[j]],
                    rows_v[b].at[pl.ds(j * CHUNK, CHUNK)],
                    gsem[b],
                ).wait()

        def start_out(g, b):
            grow = row_base + g * K
            pltpu.async_copy(
                rows_v[b], out_hbm.at[pl.ds(grow * CHUNK, GROUP)], osem[b]
            )

        def wait_out(b):
            pltpu.make_async_copy(
                rows_v[b], out_hbm.at[pl.ds(0, GROUP)], osem[b]
            ).wait()

        fire_group(0, 0)

        def pair_body(p, carry):
            @pl.when(p > 0)
            def _():
                wait_out(1)

            fire_group(2 * p + 1, 1)
            drain_gathers(0)
            start_out(2 * p, 0)
            wait_out(0)

            @pl.when(p < n_pairs - 1)
            def _():
                fire_group(2 * p + 2, 0)

            drain_gathers(1)
            start_out(2 * p + 1, 1)
            return carry

        lax.fori_loop(0, n_pairs, pair_body, 0)
        wait_out(1)

    return gather_kernel


def kernel(input_ids, table):
    B, H = input_ids.shape
    V, D = table.shape
    N = B * H
    info = plsc.get_sparse_core_info()
    NC, NS = info.num_cores, info.num_subcores
    idx2d = input_ids.reshape(N // 128, 128)
    gather = _make_gather(N, V, D, NC, NS)
    out = gather(idx2d, table)
    return out.reshape(B, H, D)


# speedup vs baseline: 4.9528x; 1.0004x over previous
import functools

import jax
import jax.numpy as jnp
from jax import lax
from jax.experimental import pallas as pl
from jax.experimental.pallas import tpu as pltpu
from jax.experimental.pallas import tpu_sc as plsc


def _make_gather(N, V, D, NC, NS):
    NW = NC * NS
    CHUNK = 512
    K = 2
    GROUP = K * CHUNK
    n_per_w = N // NW
    n_groups = n_per_w // GROUP
    assert N % (NW * GROUP) == 0 and n_groups % 2 == 0
    n_pairs = n_groups // 2

    mesh = plsc.VectorSubcoreMesh(core_axis_name="c", subcore_axis_name="s")

    @functools.partial(
        pl.kernel,
        mesh=mesh,
        out_type=jax.ShapeDtypeStruct((N, D), jnp.float32),
        scratch_types=[
            pltpu.VMEM((K, CHUNK), jnp.int32),
            pltpu.VMEM((K, CHUNK), jnp.int32),
            pltpu.VMEM((GROUP, D), jnp.float32),
            pltpu.VMEM((GROUP, D), jnp.float32),
            pltpu.SemaphoreType.DMA,
            pltpu.SemaphoreType.DMA,
            pltpu.SemaphoreType.DMA,
            pltpu.SemaphoreType.DMA,
        ],
        compiler_params=pltpu.CompilerParams(use_tc_tiling_on_sc=False),
    )
    def gather_kernel(
        idx_hbm, table_hbm, out_hbm,
        idx_v0, idx_v1, rows_v0, rows_v1, gsem0, gsem1, osem0, osem1,
    ):
        wid = lax.axis_index("s") * NC + lax.axis_index("c")
        row_base = wid * (n_per_w // CHUNK)
        idx_v = (idx_v0, idx_v1)
        rows_v = (rows_v0, rows_v1)
        gsem = (gsem0, gsem1)
        osem = (osem0, osem1)

        def fire_group(g, b):
            grow = row_base + g * K
            pltpu.sync_copy(idx_hbm.at[pl.ds(grow, K)], idx_v[b])
            for j in range(K):
                pltpu.async_copy(
                    table_hbm.at[idx_v[b].at[j]],
                    rows_v[b].at[pl.ds(j * CHUNK, CHUNK)],
                    gsem[b],
                )

        def drain_gathers(b):
            for j in range(K):
                pltpu.make_async_copy(
                    table_hbm.at[idx_v[b].at[j]],
                    rows_v[b].at[pl.ds(j * CHUNK, CHUNK)],
                    gsem[b],
                ).wait()

        def start_out(g, b):
            grow = row_base + g * K
            pltpu.async_copy(
                rows_v[b], out_hbm.at[pl.ds(grow * CHUNK, GROUP)], osem[b]
            )

        def wait_out(b):
            pltpu.make_async_copy(
                rows_v[b], out_hbm.at[pl.ds(0, GROUP)], osem[b]
            ).wait()

        fire_group(0, 0)

        def pair_body(p, carry):
            @pl.when(p > 0)
            def _():
                wait_out(1)

            fire_group(2 * p + 1, 1)
            drain_gathers(0)
            start_out(2 * p, 0)
            wait_out(0)

            @pl.when(p < n_pairs - 1)
            def _():
                fire_group(2 * p + 2, 0)

            drain_gathers(1)
            start_out(2 * p + 1, 1)
            return carry

        lax.fori_loop(0, n_pairs, pair_body, 0)
        wait_out(1)

    return gather_kernel


def kernel(input_ids, table):
    B, H = input_ids.shape
    V, D = table.shape
    N = B * H
    info = plsc.get_sparse_core_info()
    NC, NS = info.num_cores, info.num_subcores
    idx2d = input_ids.reshape(N // 512, 512)
    gather = _make_gather(N, V, D, NC, NS)
    out = gather(idx2d, table)
    return out.reshape(B, H, D)
